# baseline (speedup 1.0000x reference)
import dataclasses
import functools

import jax
import jax.numpy as jnp
from jax import lax
from jax.experimental import pallas as pl
from jax.experimental.pallas import tpu as pltpu
from jax.experimental.pallas import tpu_sc as plsc

_F = 26
_V = 100000
_D = 16
_FD = _F * _D
_BT = 1024

_NC = 2
_NS = 16
_NW = _NC * _NS
_CH = 128
_WROWS = (_F * _V + 127) // 128 + 1


def _sc_compiler_params():
    cp = pltpu.CompilerParams()
    if "needs_layout_passes" in pltpu.CompilerParams.__dataclass_fields__:
        cp = dataclasses.replace(cp, needs_layout_passes=False)
    return cp


def _sc_gather(eidx8, epm, widx, wsel, tblw, w128):
    n = eidx8.shape[0]
    per_w = n // _NW
    n_chunks = per_w // _CH
    mesh = plsc.VectorSubcoreMesh(core_axis_name="c", subcore_axis_name="s")

    @functools.partial(
        pl.kernel,
        mesh=mesh,
        out_type=(jax.ShapeDtypeStruct((n, _D), jnp.float32),
                  jax.ShapeDtypeStruct((n,), jnp.float32)),
        scratch_types=[
            pltpu.VMEM((_CH,), jnp.int32),
            pltpu.VMEM((_CH,), jnp.int32),
            pltpu.VMEM((_CH,), jnp.int32),
            pltpu.VMEM((_CH,), jnp.int32),
            pltpu.VMEM((_CH, 128), jnp.float32),
            pltpu.VMEM((_CH, 128), jnp.float32),
            pltpu.VMEM((_CH, _D), jnp.float32),
            pltpu.VMEM((_CH,), jnp.float32),
            pltpu.SemaphoreType.DMA,
            pltpu.SemaphoreType.DMA,
        ],
        compiler_params=_sc_compiler_params(),
    )
    def kern(tblw_hbm, w128_hbm, eidx_hbm, epm_hbm, widx_hbm, wsel_hbm,
             e_out, w_out,
             eidx_v, epm_v, widx_v, wsel_v, erows_v, wrows_v, eo_v, wo_v,
             sem_e, sem_w):
        wid = lax.axis_index("s") * _NC + lax.axis_index("c")

        @pl.loop(0, n_chunks)
        def _(c):
            base = wid * per_w + c * _CH
            pltpu.sync_copy(eidx_hbm.at[pl.ds(base, _CH)], eidx_v)
            pltpu.sync_copy(epm_hbm.at[pl.ds(base, _CH)], epm_v)
            pltpu.sync_copy(widx_hbm.at[pl.ds(base, _CH)], widx_v)
            pltpu.sync_copy(wsel_hbm.at[pl.ds(base, _CH)], wsel_v)
            cp_e = pltpu.async_copy(tblw_hbm.at[eidx_v], erows_v, sem_e)
            cp_w = pltpu.async_copy(w128_hbm.at[widx_v], wrows_v, sem_w)
            cp_e.wait()
            cp_w.wait()
            iota = lax.iota(jnp.int32, 16)
            for r0 in range(0, _CH, 16):
                rows = iota + r0
                epm16 = epm_v[pl.ds(r0, 16)]
                for d in range(_D):
                    vals = plsc.load_gather(erows_v, [rows, epm16 + d])
                    plsc.store_scatter(
                        eo_v, [rows, jnp.full((16,), d, jnp.int32)], vals)
                wsel16 = wsel_v[pl.ds(r0, 16)]
                wvals = plsc.load_gather(wrows_v, [rows, wsel16])
                wo_v[pl.ds(r0, 16)] = wvals
            pltpu.sync_copy(eo_v, e_out.at[pl.ds(base, _CH)])
            pltpu.sync_copy(wo_v, w_out.at[pl.ds(base, _CH)])

    return kern(tblw, w128, eidx8, epm, widx, wsel)


def _afm_body(e_ref, wv_ref, wrep_ref, hrep_ref, prep_ref, bt_ref, bias_ref,
              o_ref):
    Eb = e_ref[...].astype(jnp.bfloat16)
    Wrep = wrep_ref[...]
    Hrep = hrep_ref[...]
    Prep = prep_ref[...]
    bt = bt_ref[...]

    scores = []
    svals = []
    for i in range(_F):
        ei = Eb[:, i * _D:(i + 1) * _D]
        inter = jnp.concatenate([ei] * _F, axis=1) * Eb
        z = jnp.dot(inter, Wrep, preferred_element_type=jnp.float32)
        h = jnp.maximum(z + bt, 0.0).astype(jnp.bfloat16)
        scores.append(jnp.dot(h, Hrep, preferred_element_type=jnp.float32))
        svals.append(jnp.dot(inter, Prep, preferred_element_type=jnp.float32))

    s = jnp.concatenate(scores, axis=1)
    v = jnp.concatenate(svals, axis=1)
    lane = jax.lax.broadcasted_iota(jnp.int32, s.shape, 1)
    s = jnp.where(lane % (_F + 1) == 0, -jnp.inf, s)
    m = jnp.max(s, axis=1, keepdims=True)
    es = jnp.exp(s - m)
    att = (jnp.sum(es * v, axis=1, keepdims=True)
           / jnp.sum(es, axis=1, keepdims=True))
    first = jnp.sum(wv_ref[...], axis=1, keepdims=True)
    o_ref[...] = jax.nn.sigmoid(bias_ref[...] + first + att)


def _tc_attention(e2d, wv2d, wrep, hrep, prep, btile, bias2d):
    b = e2d.shape[0]
    return pl.pallas_call(
        _afm_body,
        grid=(b // _BT,),
        in_specs=[
            pl.BlockSpec((_BT, _FD), lambda i: (i, 0)),
            pl.BlockSpec((_BT, _F), lambda i: (i, 0)),
            pl.BlockSpec((_FD, _FD), lambda i: (0, 0)),
            pl.BlockSpec((_FD, _F), lambda i: (0, 0)),
            pl.BlockSpec((_FD, _F), lambda i: (0, 0)),
            pl.BlockSpec((1, _FD), lambda i: (0, 0)),
            pl.BlockSpec((1, 1), lambda i: (0, 0)),
        ],
        out_specs=pl.BlockSpec((_BT, 1), lambda i: (i, 0)),
        out_shape=jax.ShapeDtypeStruct((b, 1), jnp.float32),
    )(e2d, wv2d, wrep, hrep, prep, btile, bias2d)


def kernel(inputs, emb_tables, w_first, att_W, att_b, att_h, att_p, bias):
    b, f = inputs.shape
    offs = (jnp.arange(f, dtype=inputs.dtype) * _V)[None, :]
    flat = (inputs + offs).reshape(b * f)
    eidx8 = flat // 8
    epm = (flat % 8) * _D
    widx = flat // 128
    wsel = flat % 128
    tblw = emb_tables.reshape(f * _V // 8, 128)
    wpad = jnp.pad(w_first.reshape(f * _V), (0, _WROWS * 128 - f * _V))
    w128 = wpad.reshape(_WROWS, 128)

    egath, wvals = _sc_gather(eidx8, epm, widx, wsel, tblw, w128)
    e2d = egath.reshape(b, f * _D)
    wv2d = wvals.reshape(b, f)

    eye = jnp.eye(f, dtype=jnp.float32)
    wrep = jnp.kron(eye, att_W).astype(jnp.bfloat16)
    hrep = jnp.kron(eye, att_h[:, None]).astype(jnp.bfloat16)
    prep = jnp.kron(eye, att_p[:, None]).astype(jnp.bfloat16)
    btile = jnp.tile(att_b, f)[None, :]
    bias2d = bias.reshape(1, 1)

    return _tc_attention(e2d, wv2d, wrep, hrep, prep, btile, bias2d)
